# baseline (speedup 1.0000x reference)
import jax
import jax.numpy as jnp
from jax.experimental import pallas as pl
from jax.experimental.pallas import tpu as pltpu

_RADIUS = 0.1


def _fused_body(pos_ref, posT_ref, a2_ref, b2_ref, data_ref, weight_ref, out_ref):
    cross = jax.lax.dot_general(pos_ref[...], posT_ref[...],
                                (((1,), (0,)), ((), ())),
                                preferred_element_type=jnp.float32)
    base = jax.lax.dot_general(a2_ref[...], b2_ref[...],
                               (((1,), (0,)), ((), ())),
                               preferred_element_type=jnp.float32,
                               precision=jax.lax.Precision.HIGHEST)
    d2 = jnp.maximum(base - 2.0 * cross, 0.0)
    dist = jnp.sqrt(d2)
    t = jnp.maximum(1.0 - dist * (1.0 / _RADIUS), 0.0)
    w = t * t * t
    wd = jnp.dot(weight_ref[...], data_ref[...],
                 preferred_element_type=jnp.float32,
                 precision=jax.lax.Precision.HIGHEST)
    out_ref[...] = jax.lax.dot_general(
        wd, w, (((1,), (1,)), ((), ())),
        preferred_element_type=jnp.float32,
        precision=jax.lax.Precision.HIGHEST)


def kernel(locs, data, density, weight, bias):
    B, N, _ = locs.shape
    C_in = data.shape[1]
    C_out = weight.shape[0]
    pos = locs[0, :, :3]
    zcol = jnp.zeros((N, 1), jnp.float32)
    pos4 = jnp.concatenate([pos, zcol], axis=1)
    p2 = jnp.sum(pos * pos, axis=1, keepdims=True)
    ones = jnp.ones((N, 1), jnp.float32)
    zeros6 = jnp.zeros((N, 6), jnp.float32)
    a2 = jnp.concatenate([p2, ones, zeros6], axis=1)
    b2 = jnp.concatenate([ones, p2, zeros6], axis=1).T
    data2 = data[0] / density.reshape(1, N)
    w2 = weight[:, :, 0]

    BI = 256
    grid = (N // BI,)
    out = pl.pallas_call(
        _fused_body,
        grid=grid,
        in_specs=[
            pl.BlockSpec((BI, 4), lambda i: (i, 0)),
            pl.BlockSpec((4, N), lambda i: (0, 0)),
            pl.BlockSpec((BI, 8), lambda i: (i, 0)),
            pl.BlockSpec((8, N), lambda i: (0, 0)),
            pl.BlockSpec((C_in, N), lambda i: (0, 0)),
            pl.BlockSpec((C_out, C_in), lambda i: (0, 0)),
        ],
        out_specs=pl.BlockSpec((C_out, BI), lambda i: (0, i)),
        out_shape=jax.ShapeDtypeStruct((C_out, N), jnp.float32),
    )(pos4, pos4.T, a2, b2, data2, w2)
    return (out + bias.reshape(C_out, 1)).reshape(B, C_out, N)


# speedup vs baseline: 4.2241x; 4.2241x over previous
import jax
import jax.numpy as jnp
from jax.experimental import pallas as pl
from jax.experimental.pallas import tpu as pltpu

_RADIUS = 0.1


def _fused_body(pos_ref, posT_ref, p2c_ref, p2r_ref, data_ref, weight_ref,
                out_ref, wd_ref):
    @pl.when(pl.program_id(0) == 0)
    def _():
        wd_ref[...] = jnp.dot(weight_ref[...], data_ref[...],
                              preferred_element_type=jnp.float32,
                              precision=jax.lax.Precision.HIGHEST)

    cross = jax.lax.dot_general(pos_ref[...], posT_ref[...],
                                (((1,), (0,)), ((), ())),
                                preferred_element_type=jnp.float32)
    d2 = jnp.maximum((p2c_ref[...] + p2r_ref[...]) - 2.0 * cross, 0.0)
    dist = jnp.sqrt(d2)
    t = jnp.maximum(1.0 - dist * (1.0 / _RADIUS), 0.0)
    w = t * t * t
    out_ref[...] = jax.lax.dot_general(
        wd_ref[...], w, (((1,), (1,)), ((), ())),
        preferred_element_type=jnp.float32)


def kernel(locs, data, density, weight, bias):
    B, N, _ = locs.shape
    C_in = data.shape[1]
    C_out = weight.shape[0]
    pos = locs[0, :, :3]
    zcol = jnp.zeros((N, 1), jnp.float32)
    pos4 = jnp.concatenate([pos, zcol], axis=1)
    p2 = jnp.sum(pos * pos, axis=1, keepdims=True)
    data2 = data[0] / density.reshape(1, N)
    w2 = weight[:, :, 0]

    BI = 256
    grid = (N // BI,)
    out = pl.pallas_call(
        _fused_body,
        grid=grid,
        in_specs=[
            pl.BlockSpec((BI, 4), lambda i: (i, 0)),
            pl.BlockSpec((4, N), lambda i: (0, 0)),
            pl.BlockSpec((BI, 1), lambda i: (i, 0)),
            pl.BlockSpec((1, N), lambda i: (0, 0)),
            pl.BlockSpec((C_in, N), lambda i: (0, 0)),
            pl.BlockSpec((C_out, C_in), lambda i: (0, 0)),
        ],
        out_specs=pl.BlockSpec((C_out, BI), lambda i: (0, i)),
        out_shape=jax.ShapeDtypeStruct((C_out, N), jnp.float32),
        scratch_shapes=[pltpu.VMEM((C_out, N), jnp.float32)],
    )(pos4, pos4.T, p2, p2.T, data2, w2)
    return (out + bias.reshape(C_out, 1)).reshape(B, C_out, N)
